# baseline (speedup 1.0000x reference)
import functools
import jax
import jax.numpy as jnp
from jax import lax
from jax.experimental import pallas as pl
from jax.experimental.pallas import tpu as pltpu, tpu_sc as plsc

VOCAB_D = 1024
SEQ_LEN = 2048
N_BATCH = 16
P_LEN = 40
N_REL1 = 238

C = 32
N_CHUNK = 1024 // C
DEPTH = 3


def _body(tokens_hbm, wte_hbm, prompts_hbm, out_hbm,
          idx_v, buf0, buf1, buf2, tok16, idxpa, idxpb,
          gs0, gs1, gs2, ws0, ws1, ws2):
    p = lax.axis_index("c")
    b = lax.axis_index("s")
    base = b * SEQ_LEN + P_LEN + p * (1024 - P_LEN)

    bufs = (buf0, buf1, buf2)
    gss = (gs0, gs1, gs2)
    wss = (ws0, ws1, ws2)

    pltpu.sync_copy(tokens_hbm.at[pl.ds(base, 1024)], idx_v)

    def fire_gather(i):
        s = i % DEPTH
        pltpu.async_copy(wte_hbm.at[idx_v.at[pl.ds(i * C, C)]],
                         bufs[s], gss[s])

    def wait_gather(i):
        s = i % DEPTH
        pltpu.make_async_copy(wte_hbm.at[idx_v.at[pl.ds(i * C, C)]],
                              bufs[s], gss[s]).wait()

    def fire_write(i):
        s = i % DEPTH
        pltpu.async_copy(bufs[s], out_hbm.at[pl.ds(base + i * C, C)], wss[s])

    def wait_write(i):
        s = i % DEPTH
        pltpu.make_async_copy(bufs[s], out_hbm.at[pl.ds(base + i * C, C)],
                              wss[s]).wait()

    @pl.when(p == 0)
    def _prompt_phase():
        pltpu.sync_copy(tokens_hbm.at[pl.ds(b * SEQ_LEN + P_LEN, 16)], tok16)
        tv = tok16[pl.ds(0, 16)]
        r = (tv[1] % N_REL1) * P_LEN
        io = lax.iota(jnp.int32, 16)
        idxpa[pl.ds(0, 16)] = io + r
        idxpa[pl.ds(16, 16)] = io + (r + 16)
        idxpb[pl.ds(0, 16)] = jnp.minimum(io + 32, P_LEN - 1) + r
        idxpb[pl.ds(16, 16)] = jnp.minimum(io + 48, P_LEN - 1) + r
        pltpu.async_copy(prompts_hbm.at[idxpa], buf0, gs0)
        pltpu.async_copy(prompts_hbm.at[idxpb], buf1, gs1)
        pltpu.make_async_copy(prompts_hbm.at[idxpa], buf0, gs0).wait()
        pltpu.make_async_copy(prompts_hbm.at[idxpb], buf1, gs1).wait()
        pltpu.sync_copy(buf0, out_hbm.at[pl.ds(b * SEQ_LEN, C)])
        pltpu.sync_copy(buf1.at[pl.ds(0, P_LEN - C)],
                        out_hbm.at[pl.ds(b * SEQ_LEN + C, P_LEN - C)])

    for i in range(N_CHUNK):
        if i >= DEPTH:
            wait_write(i - DEPTH)
        fire_gather(i)
        if i >= 2:
            wait_gather(i - 2)
            fire_write(i - 2)
    for i in range(N_CHUNK - 2, N_CHUNK):
        wait_gather(i)
        fire_write(i)
    for i in range(N_CHUNK - DEPTH, N_CHUNK):
        wait_write(i)


@functools.partial(
    pl.kernel,
    out_type=jax.ShapeDtypeStruct((N_BATCH * SEQ_LEN, VOCAB_D), jnp.float32),
    mesh=plsc.VectorSubcoreMesh(core_axis_name="c", subcore_axis_name="s"),
    scratch_types=[
        pltpu.VMEM((1024,), jnp.int32),
        pltpu.VMEM((C, VOCAB_D), jnp.float32),
        pltpu.VMEM((C, VOCAB_D), jnp.float32),
        pltpu.VMEM((C, VOCAB_D), jnp.float32),
        pltpu.VMEM((16,), jnp.int32),
        pltpu.VMEM((32,), jnp.int32),
        pltpu.VMEM((32,), jnp.int32),
        pltpu.SemaphoreType.DMA,
        pltpu.SemaphoreType.DMA,
        pltpu.SemaphoreType.DMA,
        pltpu.SemaphoreType.DMA,
        pltpu.SemaphoreType.DMA,
        pltpu.SemaphoreType.DMA,
    ],
)
def _gather_kernel(tokens_hbm, wte_hbm, prompts_hbm, out_hbm, *scratch):
    _body(tokens_hbm, wte_hbm, prompts_hbm, out_hbm, *scratch)


@jax.jit
def kernel(tokens, wte_weight, prompts):
    tokens_flat = tokens.reshape(-1)
    prompts_flat = prompts.reshape(N_REL1 * P_LEN, VOCAB_D)
    out = _gather_kernel(tokens_flat, wte_weight, prompts_flat)
    return out.reshape(N_BATCH, SEQ_LEN, VOCAB_D)


# speedup vs baseline: 1.0068x; 1.0068x over previous
import functools
import jax
import jax.numpy as jnp
from jax import lax
from jax.experimental import pallas as pl
from jax.experimental.pallas import tpu as pltpu, tpu_sc as plsc

VOCAB_D = 1024
SEQ_LEN = 2048
N_BATCH = 16
P_LEN = 40
N_REL1 = 238

C = 48
CHUNKS = [(i * C, C) for i in range(21)] + [(21 * C, 1024 - 21 * C)]


def _body(tokens_hbm, wte_hbm, prompts_hbm, out_hbm,
          idx_v, buf0, buf1, tok16, idxp,
          gs0, gs1):
    p = lax.axis_index("c")
    b = lax.axis_index("s")
    base = b * SEQ_LEN + P_LEN + p * (1024 - P_LEN)

    bufs = (buf0, buf1)
    gss = (gs0, gs1)

    pltpu.sync_copy(tokens_hbm.at[pl.ds(base, 1024)], idx_v)

    def fire(i):
        off, cnt = CHUNKS[i]
        s = i % 2
        pltpu.async_copy(wte_hbm.at[idx_v.at[pl.ds(off, cnt)]],
                         bufs[s].at[pl.ds(0, cnt)], gss[s])

    def wait_and_write(i):
        off, cnt = CHUNKS[i]
        s = i % 2
        pltpu.make_async_copy(wte_hbm.at[idx_v.at[pl.ds(off, cnt)]],
                              bufs[s].at[pl.ds(0, cnt)], gss[s]).wait()
        pltpu.sync_copy(bufs[s].at[pl.ds(0, cnt)],
                        out_hbm.at[pl.ds(base + off, cnt)])

    @pl.when(p == 0)
    def _prompt_phase():
        pltpu.sync_copy(tokens_hbm.at[pl.ds(b * SEQ_LEN + P_LEN, 16)], tok16)
        tv = tok16[pl.ds(0, 16)]
        r = (tv[1] % N_REL1) * P_LEN
        io = lax.iota(jnp.int32, 16)
        for k in range(3):
            idxp[pl.ds(16 * k, 16)] = jnp.minimum(io + 16 * k, P_LEN - 1) + r
        pltpu.async_copy(prompts_hbm.at[idxp], buf0, gs0).wait()
        pltpu.sync_copy(buf0.at[pl.ds(0, P_LEN)],
                        out_hbm.at[pl.ds(b * SEQ_LEN, P_LEN)])

    fire(0)
    for i in range(len(CHUNKS)):
        if i + 1 < len(CHUNKS):
            fire(i + 1)
        wait_and_write(i)


@functools.partial(
    pl.kernel,
    out_type=jax.ShapeDtypeStruct((N_BATCH * SEQ_LEN, VOCAB_D), jnp.float32),
    mesh=plsc.VectorSubcoreMesh(core_axis_name="c", subcore_axis_name="s"),
    scratch_types=[
        pltpu.VMEM((1024,), jnp.int32),
        pltpu.VMEM((C, VOCAB_D), jnp.float32),
        pltpu.VMEM((C, VOCAB_D), jnp.float32),
        pltpu.VMEM((16,), jnp.int32),
        pltpu.VMEM((48,), jnp.int32),
        pltpu.SemaphoreType.DMA,
        pltpu.SemaphoreType.DMA,
    ],
)
def _gather_kernel(tokens_hbm, wte_hbm, prompts_hbm, out_hbm, *scratch):
    _body(tokens_hbm, wte_hbm, prompts_hbm, out_hbm, *scratch)


@jax.jit
def kernel(tokens, wte_weight, prompts):
    tokens_flat = tokens.reshape(-1)
    prompts_flat = prompts.reshape(N_REL1 * P_LEN, VOCAB_D)
    out = _gather_kernel(tokens_flat, wte_weight, prompts_flat)
    return out.reshape(N_BATCH, SEQ_LEN, VOCAB_D)


# speedup vs baseline: 1.0430x; 1.0360x over previous
import functools
import jax
import jax.numpy as jnp
from jax import lax
from jax.experimental import pallas as pl
from jax.experimental.pallas import tpu as pltpu, tpu_sc as plsc

VOCAB_D = 1024
SEQ_LEN = 2048
N_BATCH = 16
P_LEN = 40
N_REL1 = 238
HALF_P = P_LEN // 2

C = 32
N_CHUNK = 1024 // C


def _body(tokens_hbm, wte_hbm, prompts_hbm, out_hbm,
          idx_v, buf0, buf1, tok16, idxp, pbuf,
          gs0, gs1, psem):
    p = lax.axis_index("c")
    b = lax.axis_index("s")
    base = b * SEQ_LEN + P_LEN + p * (1024 - P_LEN)

    pltpu.sync_copy(tokens_hbm.at[pl.ds(base, 1024)], idx_v)

    def start_chunk(i, buf, sem):
        pltpu.async_copy(wte_hbm.at[idx_v.at[pl.ds(i * C, C)]], buf, sem)

    def wait_chunk(i, buf, sem):
        pltpu.make_async_copy(wte_hbm.at[idx_v.at[pl.ds(i * C, C)]],
                              buf, sem).wait()

    start_chunk(0, buf0, gs0)

    pltpu.sync_copy(tokens_hbm.at[pl.ds(b * SEQ_LEN + P_LEN, 16)], tok16)
    tv = tok16[pl.ds(0, 16)]
    r = (tv[1] % N_REL1) * P_LEN
    io = lax.iota(jnp.int32, 16)

    def prompt_share(pr0, cnt):
        idxp[pl.ds(0, 16)] = jnp.minimum(io, cnt - 1) + (r + pr0)
        idxp[pl.ds(16, 16)] = jnp.minimum(io + 16, cnt - 1) + (r + pr0)
        pltpu.async_copy(prompts_hbm.at[idxp], pbuf, psem).wait()
        pltpu.sync_copy(pbuf.at[pl.ds(0, cnt)],
                        out_hbm.at[pl.ds(b * SEQ_LEN + pr0, cnt)])

    @pl.when(p == 0)
    def _():
        prompt_share(0, 24)

    @pl.when(p == 1)
    def _():
        prompt_share(24, 16)

    def loop_body(j, carry):
        start_chunk(2 * j + 1, buf1, gs1)
        wait_chunk(2 * j, buf0, gs0)
        pltpu.sync_copy(buf0, out_hbm.at[pl.ds(base + (2 * j) * C, C)])

        @pl.when(j < N_CHUNK // 2 - 1)
        def _():
            start_chunk(2 * j + 2, buf0, gs0)

        wait_chunk(2 * j + 1, buf1, gs1)
        pltpu.sync_copy(buf1, out_hbm.at[pl.ds(base + (2 * j + 1) * C, C)])
        return carry

    lax.fori_loop(0, N_CHUNK // 2, loop_body, 0)


@functools.partial(
    pl.kernel,
    out_type=jax.ShapeDtypeStruct((N_BATCH * SEQ_LEN, VOCAB_D), jnp.float32),
    mesh=plsc.VectorSubcoreMesh(core_axis_name="c", subcore_axis_name="s"),
    scratch_types=[
        pltpu.VMEM((1024,), jnp.int32),
        pltpu.VMEM((C, VOCAB_D), jnp.float32),
        pltpu.VMEM((C, VOCAB_D), jnp.float32),
        pltpu.VMEM((16,), jnp.int32),
        pltpu.VMEM((32,), jnp.int32),
        pltpu.VMEM((32, VOCAB_D), jnp.float32),
        pltpu.SemaphoreType.DMA,
        pltpu.SemaphoreType.DMA,
        pltpu.SemaphoreType.DMA,
    ],
)
def _gather_kernel(tokens_hbm, wte_hbm, prompts_hbm, out_hbm, *scratch):
    _body(tokens_hbm, wte_hbm, prompts_hbm, out_hbm, *scratch)


@jax.jit
def kernel(tokens, wte_weight, prompts):
    tokens_flat = tokens.reshape(-1)
    prompts_flat = prompts.reshape(N_REL1 * P_LEN, VOCAB_D)
    out = _gather_kernel(tokens_flat, wte_weight, prompts_flat)
    return out.reshape(N_BATCH, SEQ_LEN, VOCAB_D)
